# baseline (speedup 1.0000x reference)
import jax
import jax.numpy as jnp
from jax import lax
from jax.experimental import pallas as pl
from jax.experimental.pallas import tpu as pltpu
from jax.experimental.pallas import tpu_sc as plsc

_N_ATOMS = 320000
_D = 128
_NSEG = 4096
_TILE = 128
_NT = _N_ATOMS // _TILE
_NC, _NS = 2, 16
_NW = _NC * _NS
_TPW = 80
_NBUF = 4
_RPS = _NSEG // _NS
_IDS_PAD = _NW * _TPW
_LROWS = 160
_MCH = 80


def _sc_body(feat_hbm, ids2d_hbm, part_hbm, idx_v, rows_v, lacc_v, midx_v,
             acc_sh, loadsems, scatsems, idsem):
    c = lax.axis_index("c")
    s = lax.axis_index("s")
    w = c * _NS + s
    t0 = w * _TPW
    nvalid = jnp.minimum(_TPW, _NT - t0)

    ids_copy = pltpu.make_async_copy(ids2d_hbm.at[pl.ds(t0, _TPW)], idx_v,
                                     idsem)
    ids_copy.start()

    @pl.loop(0, _TILE)
    def _zero_rows(i):
        @pl.loop(0, _D // 16)
        def _zero_vec(j):
            rows_v[0, i, pl.ds(j * 16, 16)] = jnp.zeros((16,), jnp.float32)

    pltpu.sync_copy(rows_v.at[0], acc_sh.at[pl.ds(s * _RPS, _TILE)])
    pltpu.sync_copy(rows_v.at[0], acc_sh.at[pl.ds(s * _RPS + _TILE, _TILE)])

    for b in range(_NBUF):
        pltpu.make_async_copy(
            feat_hbm.at[pl.ds((t0 + b) * _TILE, _TILE)],
            rows_v.at[b],
            loadsems.at[b],
        ).start()

    ids_copy.wait()
    plsc.subcore_barrier()

    lo = idx_v[0, pl.ds(0, 16)][0]
    hi = idx_v[nvalid - 1, pl.ds(_TILE - 16, 16)][15]
    span = hi - lo + 1
    use_local = span <= _LROWS

    def _valid(i):
        return (i < _TPW) & (t0 + i < _NT)

    def _load(i, b):
        pltpu.make_async_copy(
            feat_hbm.at[pl.ds((t0 + i) * _TILE, _TILE)],
            rows_v.at[b],
            loadsems.at[b],
        ).start()

    def _wait_load(i, b):
        pltpu.make_async_copy(
            feat_hbm.at[pl.ds((t0 + i) * _TILE, _TILE)],
            rows_v.at[b],
            loadsems.at[b],
        ).wait()

    def _scat_start(i, b, m):
        pltpu.async_copy(rows_v.at[b], acc_sh.at[idx_v.at[i]],
                         scatsems.at[m], add=True)

    def _scat_drain(i, b, m):
        pltpu.make_async_copy(rows_v.at[b], acc_sh.at[idx_v.at[i]],
                              scatsems.at[m]).wait()

    def _compute_k(io, k):
        rel16 = idx_v[io, pl.ds(k * 16, 16)] - lo
        first = rel16[0]
        last = rel16[15]

        @pl.when(first == last)
        def _uniform():
            for j in range(_D // 16):
                col = pl.ds(j * 16, 16)
                acc = [rows_v[3, k * 16 + l, col] for l in range(16)]
                while len(acc) > 1:
                    acc = [a + b for a, b in zip(acc[0::2], acc[1::2])]
                plsc.addupdate(lacc_v.at[first, col], acc[0])

        @pl.when(first != last)
        def _boundary():
            for l in range(16):
                rel = rel16[l]
                for j in range(_D // 16):
                    col = pl.ds(j * 16, 16)
                    plsc.addupdate(lacc_v.at[rel, col],
                                   rows_v[3, k * 16 + l, col])

    @pl.when(use_local)
    def _local_path():
        @pl.loop(0, _LROWS)
        def _zl(r):
            @pl.loop(0, _D // 16)
            def _zlv(j):
                lacc_v[r, pl.ds(j * 16, 16)] = jnp.zeros((16,), jnp.float32)

        @pl.loop(0, _TPW // 4)
        def _grp(g):
            T0 = g * 4
            T1 = T0 + 1
            T2 = T0 + 2
            T3 = T0 + 3

            @pl.when(_valid(T0))
            def _s0():
                _wait_load(T0, 0)
                _scat_start(T0, 0, 0)

            @pl.when((g > 0) & _valid(T2 - 4))
            def _drain_prev_s2():
                _scat_drain(T2 - 4, 2, 2)

            @pl.when((g > 0) & _valid(T2))
            def _load_s2():
                _load(T2, 2)

            @pl.when(_valid(T1))
            def _s1():
                _wait_load(T1, 1)
                _scat_start(T1, 1, 1)

            @pl.when(_valid(T3))
            def _compute_a():
                _wait_load(T3, 3)

                @pl.loop(0, 4)
                def _ka(k):
                    _compute_k(T3, k)

            @pl.when(_valid(T0))
            def _drain_s0():
                _scat_drain(T0, 0, 0)

            @pl.when(_valid(T0 + 4))
            def _load_next0():
                _load(T0 + 4, 0)

            @pl.when(_valid(T3))
            def _compute_b():
                @pl.loop(4, 8)
                def _kb(k):
                    _compute_k(T3, k)

            @pl.when(_valid(T2))
            def _s2():
                _wait_load(T2, 2)
                _scat_start(T2, 2, 2)

            @pl.when(_valid(T1))
            def _drain_s1():
                _scat_drain(T1, 1, 1)

            @pl.when(_valid(T1 + 4))
            def _load_next1():
                _load(T1 + 4, 1)

            @pl.when(_valid(T3 + 4))
            def _load_next3():
                _load(T3 + 4, 3)

        @pl.when(_valid(_TPW - 2))
        def _drain_last_s2():
            _scat_drain(_TPW - 2, 2, 2)

        @pl.loop(0, _LROWS // _MCH)
        def _mk(k):
            @pl.loop(0, _MCH // 16)
            def _mv(j):
                r = k * _MCH + j * 16 + lax.iota(jnp.int32, 16)
                midx_v[k, pl.ds(j * 16, 16)] = jnp.where(
                    r < span, r + lo, _NSEG)

        for k in range(_LROWS // _MCH):
            pltpu.sync_copy(lacc_v.at[pl.ds(k * _MCH, _MCH)],
                            acc_sh.at[midx_v.at[k]], add=True)

    @pl.when(jnp.logical_not(use_local))
    def _global_path():
        @pl.loop(0, _TPW // _NBUF)
        def _grp(g):
            for b in range(_NBUF):
                i = g * _NBUF + b

                @pl.when(_valid(i))
                def _consume():
                    _wait_load(i, b)
                    pltpu.sync_copy(rows_v.at[b], acc_sh.at[idx_v.at[i]],
                                    add=True)

                @pl.when(_valid(i + _NBUF))
                def _pref():
                    _load(i + _NBUF, b)

    plsc.subcore_barrier()
    pltpu.sync_copy(
        acc_sh.at[pl.ds(s * _RPS, _RPS)],
        part_hbm.at[c, pl.ds(s * _RPS, _RPS)],
    )


def _add_body(p_ref, o_ref):
    o_ref[...] = p_ref[0] + p_ref[1]


def kernel(atom_features, node_graph_indices):
    ids2d = node_graph_indices.astype(jnp.int32).reshape(_NT, _TILE)
    ids2d = jnp.pad(ids2d, ((0, _IDS_PAD - _NT), (0, 0)))
    mesh = plsc.VectorSubcoreMesh(core_axis_name="c", subcore_axis_name="s")
    sc_call = pl.kernel(
        _sc_body,
        out_type=jax.ShapeDtypeStruct((_NC, _NSEG, _D), jnp.float32),
        mesh=mesh,
        scratch_types=[
            pltpu.VMEM((_TPW, _TILE), jnp.int32),
            pltpu.VMEM((_NBUF, _TILE, _D), jnp.float32),
            pltpu.VMEM((_LROWS, _D), jnp.float32),
            pltpu.VMEM((_LROWS // _MCH, _MCH), jnp.int32),
            pltpu.VMEM_SHARED((_NSEG + 8, _D), jnp.float32),
            pltpu.SemaphoreType.DMA((_NBUF,)),
            pltpu.SemaphoreType.DMA((3,)),
            pltpu.SemaphoreType.DMA,
        ],
    )
    part = sc_call(atom_features, ids2d)
    return pl.pallas_call(
        _add_body,
        out_shape=jax.ShapeDtypeStruct((_NSEG, _D), jnp.float32),
    )(part)


# speedup vs baseline: 1.0634x; 1.0634x over previous
import jax
import jax.numpy as jnp
from jax import lax
from jax.experimental import pallas as pl
from jax.experimental.pallas import tpu as pltpu
from jax.experimental.pallas import tpu_sc as plsc

_N_ATOMS = 320000
_D = 128
_NSEG = 4096
_TILE = 128
_NT = _N_ATOMS // _TILE
_NC, _NS = 2, 16
_NW = _NC * _NS
_TPW = 80
_NBUF = 4
_RPS = _NSEG // _NS
_IDS_PAD = _NW * _TPW


def _sc_body(feat_hbm, ids2d_hbm, part_hbm, idx_v, rows_v, acc_sh, loadsems,
             idsem):
    c = lax.axis_index("c")
    s = lax.axis_index("s")
    w = c * _NS + s
    t0 = w * _TPW

    ids_copy = pltpu.make_async_copy(ids2d_hbm.at[pl.ds(t0, _TPW)], idx_v,
                                     idsem)
    ids_copy.start()

    @pl.loop(0, _TILE)
    def _zero_rows(i):
        @pl.loop(0, _D // 16)
        def _zero_vec(j):
            rows_v[0, i, pl.ds(j * 16, 16)] = jnp.zeros((16,), jnp.float32)

    pltpu.sync_copy(rows_v.at[0], acc_sh.at[pl.ds(s * _RPS, _TILE)])
    pltpu.sync_copy(rows_v.at[0], acc_sh.at[pl.ds(s * _RPS + _TILE, _TILE)])

    for b in range(_NBUF):
        pltpu.make_async_copy(
            feat_hbm.at[pl.ds((t0 + b) * _TILE, _TILE)],
            rows_v.at[b],
            loadsems.at[b],
        ).start()

    ids_copy.wait()
    plsc.subcore_barrier()

    @pl.loop(0, _TPW // _NBUF)
    def _grp(g):
        for b in range(_NBUF):
            i = g * _NBUF + b
            t = t0 + i

            @pl.when(t < _NT)
            def _consume():
                pltpu.make_async_copy(
                    feat_hbm.at[pl.ds(t * _TILE, _TILE)],
                    rows_v.at[b],
                    loadsems.at[b],
                ).wait()
                pltpu.sync_copy(rows_v.at[b], acc_sh.at[idx_v.at[i]], add=True)

            i2 = i + _NBUF
            t2 = t + _NBUF

            @pl.when((i2 < _TPW) & (t2 < _NT))
            def _prefetch():
                pltpu.make_async_copy(
                    feat_hbm.at[pl.ds(t2 * _TILE, _TILE)],
                    rows_v.at[b],
                    loadsems.at[b],
                ).start()

    plsc.subcore_barrier()
    pltpu.sync_copy(
        acc_sh.at[pl.ds(s * _RPS, _RPS)],
        part_hbm.at[c, pl.ds(s * _RPS, _RPS)],
    )


def _add_body(p_ref, o_ref):
    o_ref[...] = p_ref[0] + p_ref[1]


def kernel(atom_features, node_graph_indices):
    ids2d = node_graph_indices.astype(jnp.int32).reshape(_NT, _TILE)
    ids2d = jnp.pad(ids2d, ((0, _IDS_PAD - _NT), (0, 0)))
    mesh = plsc.VectorSubcoreMesh(core_axis_name="c", subcore_axis_name="s")
    sc_call = pl.kernel(
        _sc_body,
        out_type=jax.ShapeDtypeStruct((_NC, _NSEG, _D), jnp.float32),
        mesh=mesh,
        scratch_types=[
            pltpu.VMEM((_TPW, _TILE), jnp.int32),
            pltpu.VMEM((_NBUF, _TILE, _D), jnp.float32),
            pltpu.VMEM_SHARED((_NSEG, _D), jnp.float32),
            pltpu.SemaphoreType.DMA((_NBUF,)),
            pltpu.SemaphoreType.DMA,
        ],
    )
    part = sc_call(atom_features, ids2d)
    return pl.pallas_call(
        _add_body,
        out_shape=jax.ShapeDtypeStruct((_NSEG, _D), jnp.float32),
    )(part)
